# baseline (speedup 1.0000x reference)
import functools

import jax
import jax.numpy as jnp
from jax import lax
from jax.experimental import pallas as pl
from jax.experimental.pallas import tpu as pltpu
from jax.experimental.pallas import tpu_sc as plsc

_NC = 2
_NS = 16
_NW = _NC * _NS
_C = 8
_NBUF = 2


def _build(B0, B1, V, D):
    rows_per_w = B0 // _NW
    nchunk = rows_per_w // _C
    mesh = plsc.VectorSubcoreMesh(core_axis_name="c", subcore_axis_name="s")

    @functools.partial(
        pl.kernel,
        mesh=mesh,
        out_type=jax.ShapeDtypeStruct((B0, B1, D), jnp.float32),
        scratch_types=[
            pltpu.VMEM((_C, B1), jnp.int32),
            pltpu.VMEM((_C, B1), jnp.int32),
            pltpu.VMEM((_C, B1, D), jnp.float32),
            pltpu.VMEM((_C, B1, D), jnp.float32),
            pltpu.SemaphoreType.DMA,
            pltpu.SemaphoreType.DMA,
            pltpu.SemaphoreType.DMA,
            pltpu.SemaphoreType.DMA,
        ],
        compiler_params=pltpu.CompilerParams(use_tc_tiling_on_sc=False),
    )
    def k(idx_hbm, table_hbm, out_hbm, idx0, idx1, rows0, rows1, g0, g1, w0, w1):
        idx_b = (idx0, idx1)
        rows_b = (rows0, rows1)
        gs = (g0, g1)
        ws = (w0, w1)
        wid = lax.axis_index("s") * _NC + lax.axis_index("c")
        cbase = wid * nchunk

        def fire(g, b):
            ibase = (cbase + g) * _C
            pltpu.sync_copy(idx_hbm.at[pl.ds(ibase, _C)], idx_b[b])
            for j in range(_C):
                pltpu.async_copy(table_hbm.at[idx_b[b].at[j]], rows_b[b].at[j], gs[b])

        def gather_wait(b):
            pltpu.make_async_copy(out_hbm.at[pl.ds(0, _C)], rows_b[b], gs[b]).wait()

        def writeback(g, b):
            pltpu.async_copy(
                rows_b[b], out_hbm.at[pl.ds((cbase + g) * _C, _C)], ws[b]
            )

        def wb_wait(b):
            pltpu.make_async_copy(rows_b[b], out_hbm.at[pl.ds(0, _C)], ws[b]).wait()

        def body(i, carry):
            for b in range(_NBUF):
                g = _NBUF * i + b

                @pl.when(g >= _NBUF)
                def _():
                    wb_wait(b)

                fire(g, b)

                @pl.when(g >= 1)
                def _():
                    gather_wait(1 - b)
                    writeback(g - 1, 1 - b)

            return carry

        lax.fori_loop(0, nchunk // _NBUF, body, None)
        gather_wait((nchunk - 1) % _NBUF)
        writeback(nchunk - 1, (nchunk - 1) % _NBUF)
        for b in range(_NBUF):
            wb_wait(b)

    return k


def kernel(x, table):
    B0, B1 = x.shape
    V, D = table.shape
    x2 = x.astype(jnp.int32) * 2
    tt = jnp.pad(table, ((0, 0), (0, 128 - D))).reshape(2 * V, D)
    return _build(B0, B1, 2 * V, D)(x2, tt)


# speedup vs baseline: 1.3706x; 1.3706x over previous
import functools

import jax
import jax.numpy as jnp
from jax import lax
from jax.experimental import pallas as pl
from jax.experimental.pallas import tpu as pltpu
from jax.experimental.pallas import tpu_sc as plsc

_NC = 2
_NS = 16
_NW = _NC * _NS
_C = 8
_NBUF = 2


def _build(B0, B1, V, D):
    rows_per_w = B0 // _NW
    nchunk = rows_per_w // _C
    mesh = plsc.VectorSubcoreMesh(core_axis_name="c", subcore_axis_name="s")

    @functools.partial(
        pl.kernel,
        mesh=mesh,
        out_type=jax.ShapeDtypeStruct((B0, 56, 128), jnp.float32),
        scratch_types=[
            pltpu.VMEM((_C, B1), jnp.int32),
            pltpu.VMEM((_C, B1), jnp.int32),
            pltpu.VMEM((_C, B1, D), jnp.float32),
            pltpu.VMEM((_C, B1, D), jnp.float32),
            pltpu.SemaphoreType.DMA,
            pltpu.SemaphoreType.DMA,
            pltpu.SemaphoreType.DMA,
            pltpu.SemaphoreType.DMA,
        ],
        compiler_params=pltpu.CompilerParams(use_tc_tiling_on_sc=False),
    )
    def k(idx_hbm, table_hbm, out_hbm, idx0, idx1, rows0, rows1, g0, g1, w0, w1):
        idx_b = (idx0, idx1)
        rows_b = (rows0, rows1)
        gs = (g0, g1)
        ws = (w0, w1)
        wid = lax.axis_index("s") * _NC + lax.axis_index("c")
        cbase = wid * nchunk

        def fire(g, b):
            ibase = (cbase + g) * _C
            pltpu.sync_copy(idx_hbm.at[pl.ds(ibase, _C)], idx_b[b])
            for j in range(_C):
                pltpu.async_copy(table_hbm.at[idx_b[b].at[j]], rows_b[b].at[j], gs[b])

        def out_slab(g):
            return out_hbm.at[
                pl.ds((cbase + g) * _C, _C), pl.ds(0, B1), pl.ds(0, D)
            ]

        def gather_wait(b):
            pltpu.make_async_copy(out_slab(0), rows_b[b], gs[b]).wait()

        def writeback(g, b):
            pltpu.async_copy(rows_b[b], out_slab(g), ws[b])

        def wb_wait(b):
            pltpu.make_async_copy(rows_b[b], out_slab(0), ws[b]).wait()

        def body(i, carry):
            for b in range(_NBUF):
                g = _NBUF * i + b

                @pl.when(g >= _NBUF)
                def _():
                    wb_wait(b)

                fire(g, b)

                @pl.when(g >= 1)
                def _():
                    gather_wait(1 - b)
                    writeback(g - 1, 1 - b)

            return carry

        lax.fori_loop(0, nchunk // _NBUF, body, None)
        gather_wait((nchunk - 1) % _NBUF)
        writeback(nchunk - 1, (nchunk - 1) % _NBUF)
        for b in range(_NBUF):
            wb_wait(b)

    return k


def kernel(x, table):
    B0, B1 = x.shape
    V, D = table.shape
    x2 = x.astype(jnp.int32) * 2
    tt = jnp.pad(table, ((0, 0), (0, 128 - D))).reshape(2 * V, D)
    padded = _build(B0, B1, 2 * V, D)(x2, tt)
    return padded[:, :B1, :D]
